# baseline (speedup 1.0000x reference)
import numpy as np

import jax
import jax.numpy as jnp
from jax import lax
from jax.experimental import pallas as pl
from jax.experimental.pallas import tpu as pltpu

_F32 = jnp.float32
_I32 = jnp.int32

_H = 50
_W = 50
_QW = 52
_NQ = _H * _QW
_NPIX = _H * _W
_NA = 9
_PRE = 2000
_PRE_PAD = 2048
_BLK = 128
_NBLK = _PRE_PAD // _BLK
_POST = 300
_THRESH = 0.7


def _anchor_qconst():
    base = []
    for r in (0.5, 1.0, 2.0):
        for s in (8.0, 16.0, 32.0):
            hh = 16.0 * s * np.sqrt(r)
            ww = 16.0 * s * np.sqrt(1.0 / r)
            base.append([-ww / 2.0, -hh / 2.0, ww / 2.0, hh / 2.0])
    base = np.asarray(base, np.float32)
    sy = (np.arange(_H, dtype=np.float32) + 0.5) * 16.0
    sx = (np.arange(_W, dtype=np.float32) + 0.5) * 16.0
    yy, xx = np.meshgrid(sy, sx, indexing="ij")
    shifts = np.stack([xx.ravel(), yy.ravel(), xx.ravel(), yy.ravel()], axis=1)
    anch = shifts[:, None, :] + base[None, :, :]
    x1, y1 = anch[..., 0], anch[..., 1]
    x2, y2 = anch[..., 2], anch[..., 3]
    aw = x2 - x1
    ah = y2 - y1
    acx = x1 + np.float32(0.5) * aw
    acy = y1 + np.float32(0.5) * ah

    def toq(a):
        out = np.zeros((_H, _QW, _NA), np.float32)
        out[:, :_W, :] = a.reshape(_H, _W, _NA)
        return out.reshape(_NQ, _NA)

    return jnp.asarray(np.concatenate([toq(acx), toq(acy), toq(aw), toq(ah)], axis=1))


def _rpn_head_kernel(xp_ref, w9_ref, b1_ref, hw_ref, hb_ref, anc_ref, sz_ref,
                     score_ref, x1_ref, y1_ref, x2_ref, y2_ref):
    acc = jnp.zeros((_NQ, 512), _F32) + b1_ref[...]
    for ky in range(3):
        for kx in range(3):
            t = ky * 3 + kx
            off = ky * _QW + kx
            acc = acc + jnp.dot(xp_ref[pl.ds(off, _NQ), :], w9_ref[t],
                                preferred_element_type=_F32)
    mid = jnp.maximum(acc, 0.0)

    heads = jnp.dot(mid, hw_ref[...], preferred_element_type=_F32) + hb_ref[...]
    tx = heads[:, 0:9]
    ty = heads[:, 9:18]
    tw = heads[:, 18:27]
    th = heads[:, 27:36]
    s0 = heads[:, 36:45]
    s1 = heads[:, 45:54]

    acx = anc_ref[:, 0:9]
    acy = anc_ref[:, 9:18]
    aw = anc_ref[:, 18:27]
    ah = anc_ref[:, 27:36]

    cx = tx * aw + acx
    cy = ty * ah + acy
    w = jnp.exp(jnp.clip(tw, -10.0, 10.0)) * aw
    h = jnp.exp(jnp.clip(th, -10.0, 10.0)) * ah

    sz = sz_ref[...]
    bx1 = jnp.clip(cx - 0.5 * w, 0.0, sz)
    by1 = jnp.clip(cy - 0.5 * h, 0.0, sz)
    bx2 = jnp.clip(cx + 0.5 * w, 0.0, sz)
    by2 = jnp.clip(cy + 0.5 * h, 0.0, sz)

    m = jnp.maximum(s0, s1)
    e0 = jnp.exp(s0 - m)
    e1 = jnp.exp(s1 - m)
    fg = e1 / (e0 + e1)

    valid = ((bx2 - bx1) >= 16.0) & ((by2 - by1) >= 16.0)
    score_ref[...] = jnp.where(valid, fg, -1e9)
    x1_ref[...] = bx1
    y1_ref[...] = by1
    x2_ref[...] = bx2
    y2_ref[...] = by2


def _nms_kernel(cc_ref, cr_ref, out_ref, sup_ref):
    rx1 = cr_ref[0:1, :]
    ry1 = cr_ref[1:2, :]
    rx2 = cr_ref[2:3, :]
    ry2 = cr_ref[3:4, :]
    arear = (rx2 - rx1) * (ry2 - ry1)

    futsup = jnp.zeros((1, _PRE_PAD), _F32)
    for k in range(_NBLK):
        c0 = k * _BLK
        cb = cc_ref[pl.ds(c0, _BLK), :]
        cx1 = cb[:, 0:1]
        cy1 = cb[:, 1:2]
        cx2 = cb[:, 2:3]
        cy2 = cb[:, 3:4]
        areac = (cx2 - cx1) * (cy2 - cy1)
        xx1 = jnp.maximum(cx1, rx1)
        yy1 = jnp.maximum(cy1, ry1)
        xx2 = jnp.minimum(cx2, rx2)
        yy2 = jnp.minimum(cy2, ry2)
        inter = jnp.maximum(xx2 - xx1, 0.0) * jnp.maximum(yy2 - yy1, 0.0)
        iou = inter / jnp.maximum(areac + arear - inter, 1e-9)
        jj = lax.broadcasted_iota(_I32, (_BLK, _PRE_PAD), 1)
        ii = lax.broadcasted_iota(_I32, (_BLK, _PRE_PAD), 0) + c0
        srow = jnp.where((iou > _THRESH) & (jj > ii), 1.0, 0.0)

        diag = srow[:, c0:c0 + _BLK]
        sblk = futsup[0:1, c0:c0 + _BLK]
        for j in range(_BLK):
            row = diag[j:j + 1, :]
            sj = sblk[0:1, j:j + 1]
            sblk = jnp.where(sj > 0.0, sblk, jnp.maximum(sblk, row))
        sup_ref[pl.ds(k, 1), :] = sblk

        contrib = jnp.dot(1.0 - sblk, srow, preferred_element_type=_F32)
        futsup = jnp.maximum(futsup, jnp.where(contrib > 0.0, 1.0, 0.0))

    sup16 = sup_ref[...]
    fidx = (lax.broadcasted_iota(_I32, (_NBLK, _BLK), 0) * _BLK
            + lax.broadcasted_iota(_I32, (_NBLK, _BLK), 1))
    valid16 = jnp.where(fidx < _PRE, 1.0, 0.0)
    kept16 = (1.0 - sup16) * valid16
    supv16 = sup16 * valid16
    tri = jnp.where(
        lax.broadcasted_iota(_I32, (_BLK, _BLK), 0)
        <= lax.broadcasted_iota(_I32, (_BLK, _BLK), 1), 1.0, 0.0)
    ckr = jnp.dot(kept16, tri, preferred_element_type=_F32)
    csr = jnp.dot(supv16, tri, preferred_element_type=_F32)
    low = jnp.where(
        lax.broadcasted_iota(_I32, (_NBLK, _NBLK), 0)
        > lax.broadcasted_iota(_I32, (_NBLK, _NBLK), 1), 1.0, 0.0)
    ck = ckr + jnp.dot(low, ckr[:, _BLK - 1:_BLK], preferred_element_type=_F32)
    cs = csr + jnp.dot(low, csr[:, _BLK - 1:_BLK], preferred_element_type=_F32)
    nkept = jnp.sum(kept16)
    rank16 = (kept16 * (ck - 1.0) + supv16 * (nkept + cs - 1.0)
              + (1.0 - valid16) * 4096.0)
    rank = jnp.concatenate([rank16[k:k + 1, :] for k in range(_NBLK)], axis=1)

    rr = lax.broadcasted_iota(_I32, (_POST, _PRE_PAD), 0)
    onehot = jnp.where(
        jnp.broadcast_to(rank.astype(_I32), (_POST, _PRE_PAD)) == rr, 1.0, 0.0)
    out_ref[...] = jnp.dot(onehot, cc_ref[...], preferred_element_type=_F32)


def kernel(features, img_size, conv1_w, conv1_b, score_w, score_b, loc_w, loc_b):
    xh = jnp.transpose(features[0], (1, 2, 0))
    x = jnp.pad(
        jnp.pad(xh, ((1, 1), (1, 1), (0, 0))).reshape(_QW * _QW, 512),
        ((0, 2720 - _QW * _QW), (0, 0)))
    w9 = jnp.transpose(conv1_w, (2, 3, 1, 0)).reshape(9, 512, 512)
    b1 = conv1_b.reshape(1, 512)
    lw = loc_w[:, :, 0, 0]
    sw = score_w[:, :, 0, 0]
    hw = jnp.concatenate(
        [lw[0::4].T, lw[1::4].T, lw[2::4].T, lw[3::4].T, sw[0::2].T, sw[1::2].T],
        axis=1)
    hb = jnp.concatenate(
        [loc_b[0::4], loc_b[1::4], loc_b[2::4], loc_b[3::4],
         score_b[0::2], score_b[1::2]]).reshape(1, 54)
    anc = _anchor_qconst()
    sz = jnp.asarray(img_size, _F32).reshape(1, 1)

    q9 = jax.ShapeDtypeStruct((_NQ, _NA), _F32)
    score_q, qx1, qy1, qx2, qy2 = pl.pallas_call(
        _rpn_head_kernel,
        out_shape=[q9, q9, q9, q9, q9],
    )(x, w9, b1, hw, hb, anc, sz)

    def unq(a):
        return a.reshape(_H, _QW, _NA)[:, :_W, :].reshape(-1)

    scores = unq(score_q)
    boxes = jnp.stack([unq(qx1), unq(qy1), unq(qx2), unq(qy2)], axis=1)
    _, top_i = lax.top_k(scores, _PRE)
    cand = boxes[top_i]
    cc = jnp.zeros((_PRE_PAD, 4), _F32).at[:_PRE].set(cand)
    cr = cc.T

    out = pl.pallas_call(
        _nms_kernel,
        out_shape=jax.ShapeDtypeStruct((_POST, 4), _F32),
        scratch_shapes=[pltpu.VMEM((_NBLK, _BLK), _F32)],
    )(cc, cr)
    return out[None]


# speedup vs baseline: 1.8361x; 1.8361x over previous
import numpy as np

import jax
import jax.numpy as jnp
from jax import lax
from jax.experimental import pallas as pl
from jax.experimental.pallas import tpu as pltpu

_F32 = jnp.float32
_I32 = jnp.int32

_H = 50
_W = 50
_QW = 52
_NQ = _H * _QW
_NPIX = _H * _W
_NA = 9
_PRE = 2000
_PRE_PAD = 2048
_BLK = 128
_NBLK = _PRE_PAD // _BLK
_POST = 300
_THRESH = 0.7
_JACOBI = 10


def _anchor_qconst():
    base = []
    for r in (0.5, 1.0, 2.0):
        for s in (8.0, 16.0, 32.0):
            hh = 16.0 * s * np.sqrt(r)
            ww = 16.0 * s * np.sqrt(1.0 / r)
            base.append([-ww / 2.0, -hh / 2.0, ww / 2.0, hh / 2.0])
    base = np.asarray(base, np.float32)
    sy = (np.arange(_H, dtype=np.float32) + 0.5) * 16.0
    sx = (np.arange(_W, dtype=np.float32) + 0.5) * 16.0
    yy, xx = np.meshgrid(sy, sx, indexing="ij")
    shifts = np.stack([xx.ravel(), yy.ravel(), xx.ravel(), yy.ravel()], axis=1)
    anch = shifts[:, None, :] + base[None, :, :]
    x1, y1 = anch[..., 0], anch[..., 1]
    x2, y2 = anch[..., 2], anch[..., 3]
    aw = x2 - x1
    ah = y2 - y1
    acx = x1 + np.float32(0.5) * aw
    acy = y1 + np.float32(0.5) * ah

    def toq(a):
        out = np.zeros((_H, _QW, _NA), np.float32)
        out[:, :_W, :] = a.reshape(_H, _W, _NA)
        return out.reshape(_NQ, _NA)

    return jnp.asarray(np.concatenate([toq(acx), toq(acy), toq(aw), toq(ah)], axis=1))


def _rpn_head_kernel(xp_ref, w9_ref, b1_ref, hw_ref, hb_ref, anc_ref, sz_ref,
                     score_ref, x1_ref, y1_ref, x2_ref, y2_ref):
    acc = jnp.zeros((_NQ, 512), _F32) + b1_ref[...]
    for ky in range(3):
        for kx in range(3):
            t = ky * 3 + kx
            off = ky * _QW + kx
            acc = acc + jnp.dot(xp_ref[pl.ds(off, _NQ), :], w9_ref[t],
                                preferred_element_type=_F32)
    mid = jnp.maximum(acc, 0.0)

    heads = jnp.dot(mid, hw_ref[...], preferred_element_type=_F32) + hb_ref[...]
    tx = heads[:, 0:9]
    ty = heads[:, 9:18]
    tw = heads[:, 18:27]
    th = heads[:, 27:36]
    s0 = heads[:, 36:45]
    s1 = heads[:, 45:54]

    acx = anc_ref[:, 0:9]
    acy = anc_ref[:, 9:18]
    aw = anc_ref[:, 18:27]
    ah = anc_ref[:, 27:36]

    cx = tx * aw + acx
    cy = ty * ah + acy
    w = jnp.exp(jnp.clip(tw, -10.0, 10.0)) * aw
    h = jnp.exp(jnp.clip(th, -10.0, 10.0)) * ah

    sz = sz_ref[...]
    bx1 = jnp.clip(cx - 0.5 * w, 0.0, sz)
    by1 = jnp.clip(cy - 0.5 * h, 0.0, sz)
    bx2 = jnp.clip(cx + 0.5 * w, 0.0, sz)
    by2 = jnp.clip(cy + 0.5 * h, 0.0, sz)

    m = jnp.maximum(s0, s1)
    e0 = jnp.exp(s0 - m)
    e1 = jnp.exp(s1 - m)
    fg = e1 / (e0 + e1)

    valid = ((bx2 - bx1) >= 16.0) & ((by2 - by1) >= 16.0)
    score_ref[...] = jnp.where(valid, fg, -1e9)
    x1_ref[...] = bx1
    y1_ref[...] = by1
    x2_ref[...] = bx2
    y2_ref[...] = by2


def _nms_kernel(cc_ref, cr_ref, out_ref, sup_ref):
    rx1 = cr_ref[0:1, :]
    ry1 = cr_ref[1:2, :]
    rx2 = cr_ref[2:3, :]
    ry2 = cr_ref[3:4, :]
    arear = (rx2 - rx1) * (ry2 - ry1)

    futsup = jnp.zeros((1, _PRE_PAD), _F32)
    for k in range(_NBLK):
        c0 = k * _BLK
        cb = cc_ref[pl.ds(c0, _BLK), :]
        cx1 = cb[:, 0:1]
        cy1 = cb[:, 1:2]
        cx2 = cb[:, 2:3]
        cy2 = cb[:, 3:4]
        areac = (cx2 - cx1) * (cy2 - cy1)
        xx1 = jnp.maximum(cx1, rx1)
        yy1 = jnp.maximum(cy1, ry1)
        xx2 = jnp.minimum(cx2, rx2)
        yy2 = jnp.minimum(cy2, ry2)
        inter = jnp.maximum(xx2 - xx1, 0.0) * jnp.maximum(yy2 - yy1, 0.0)
        iou = inter / jnp.maximum(areac + arear - inter, 1e-9)
        jj = lax.broadcasted_iota(_I32, (_BLK, _PRE_PAD), 1)
        ii = lax.broadcasted_iota(_I32, (_BLK, _PRE_PAD), 0) + c0
        srow = jnp.where((iou > _THRESH) & (jj > ii), 1.0, 0.0)

        diag = srow[:, c0:c0 + _BLK]
        ext = futsup[0:1, c0:c0 + _BLK]
        sup = ext
        for _ in range(_JACOBI):
            hit = jnp.dot(1.0 - sup, diag, preferred_element_type=_F32)
            sup = jnp.maximum(ext, jnp.where(hit > 0.0, 1.0, 0.0))
        hit = jnp.dot(1.0 - sup, diag, preferred_element_type=_F32)
        sup2 = jnp.maximum(ext, jnp.where(hit > 0.0, 1.0, 0.0))
        delta = jnp.sum(jnp.abs(sup2 - sup))

        @pl.when(delta == 0.0)
        def _():
            sup_ref[pl.ds(k, 1), :] = sup2

        @pl.when(delta != 0.0)
        def _():
            sblk = ext
            for j in range(_BLK):
                row = diag[j:j + 1, :]
                sj = sblk[0:1, j:j + 1]
                sblk = jnp.where(sj > 0.0, sblk, jnp.maximum(sblk, row))
            sup_ref[pl.ds(k, 1), :] = sblk

        sblk = sup_ref[pl.ds(k, 1), :]

        contrib = jnp.dot(1.0 - sblk, srow, preferred_element_type=_F32)
        futsup = jnp.maximum(futsup, jnp.where(contrib > 0.0, 1.0, 0.0))

    sup16 = sup_ref[...]
    fidx = (lax.broadcasted_iota(_I32, (_NBLK, _BLK), 0) * _BLK
            + lax.broadcasted_iota(_I32, (_NBLK, _BLK), 1))
    valid16 = jnp.where(fidx < _PRE, 1.0, 0.0)
    kept16 = (1.0 - sup16) * valid16
    supv16 = sup16 * valid16
    tri = jnp.where(
        lax.broadcasted_iota(_I32, (_BLK, _BLK), 0)
        <= lax.broadcasted_iota(_I32, (_BLK, _BLK), 1), 1.0, 0.0)
    ckr = jnp.dot(kept16, tri, preferred_element_type=_F32)
    csr = jnp.dot(supv16, tri, preferred_element_type=_F32)
    low = jnp.where(
        lax.broadcasted_iota(_I32, (_NBLK, _NBLK), 0)
        > lax.broadcasted_iota(_I32, (_NBLK, _NBLK), 1), 1.0, 0.0)
    ck = ckr + jnp.dot(low, ckr[:, _BLK - 1:_BLK], preferred_element_type=_F32)
    cs = csr + jnp.dot(low, csr[:, _BLK - 1:_BLK], preferred_element_type=_F32)
    nkept = jnp.sum(kept16)
    rank16 = (kept16 * (ck - 1.0) + supv16 * (nkept + cs - 1.0)
              + (1.0 - valid16) * 4096.0)
    rank = jnp.concatenate([rank16[k:k + 1, :] for k in range(_NBLK)], axis=1)

    rr = lax.broadcasted_iota(_I32, (_POST, _PRE_PAD), 0)
    onehot = jnp.where(
        jnp.broadcast_to(rank.astype(_I32), (_POST, _PRE_PAD)) == rr, 1.0, 0.0)
    out_ref[...] = jnp.dot(onehot, cc_ref[...], preferred_element_type=_F32)


def kernel(features, img_size, conv1_w, conv1_b, score_w, score_b, loc_w, loc_b):
    xh = jnp.transpose(features[0], (1, 2, 0))
    x = jnp.pad(
        jnp.pad(xh, ((1, 1), (1, 1), (0, 0))).reshape(_QW * _QW, 512),
        ((0, 2720 - _QW * _QW), (0, 0)))
    w9 = jnp.transpose(conv1_w, (2, 3, 1, 0)).reshape(9, 512, 512)
    b1 = conv1_b.reshape(1, 512)
    lw = loc_w[:, :, 0, 0]
    sw = score_w[:, :, 0, 0]
    hw = jnp.concatenate(
        [lw[0::4].T, lw[1::4].T, lw[2::4].T, lw[3::4].T, sw[0::2].T, sw[1::2].T],
        axis=1)
    hb = jnp.concatenate(
        [loc_b[0::4], loc_b[1::4], loc_b[2::4], loc_b[3::4],
         score_b[0::2], score_b[1::2]]).reshape(1, 54)
    anc = _anchor_qconst()
    sz = jnp.asarray(img_size, _F32).reshape(1, 1)

    q9 = jax.ShapeDtypeStruct((_NQ, _NA), _F32)
    score_q, qx1, qy1, qx2, qy2 = pl.pallas_call(
        _rpn_head_kernel,
        out_shape=[q9, q9, q9, q9, q9],
    )(x, w9, b1, hw, hb, anc, sz)

    def unq(a):
        return a.reshape(_H, _QW, _NA)[:, :_W, :].reshape(-1)

    scores = unq(score_q)
    boxes = jnp.stack([unq(qx1), unq(qy1), unq(qx2), unq(qy2)], axis=1)
    _, top_i = lax.top_k(scores, _PRE)
    cand = boxes[top_i]
    cc = jnp.zeros((_PRE_PAD, 4), _F32).at[:_PRE].set(cand)
    cr = cc.T

    out = pl.pallas_call(
        _nms_kernel,
        out_shape=jax.ShapeDtypeStruct((_POST, 4), _F32),
        scratch_shapes=[pltpu.VMEM((_NBLK, _BLK), _F32)],
    )(cc, cr)
    return out[None]


# speedup vs baseline: 1.9853x; 1.0812x over previous
import numpy as np

import jax
import jax.numpy as jnp
from jax import lax
from jax.experimental import pallas as pl
from jax.experimental.pallas import tpu as pltpu

_F32 = jnp.float32
_I32 = jnp.int32

_H = 50
_W = 50
_QW = 52
_NQ = _H * _QW
_NPIX = _H * _W
_NA = 9
_PRE = 2000
_PRE_PAD = 2048
_BLK = 128
_NBLK = _PRE_PAD // _BLK
_POST = 300
_THRESH = 0.7
_JACOBI = 6


def _anchor_qconst():
    base = []
    for r in (0.5, 1.0, 2.0):
        for s in (8.0, 16.0, 32.0):
            hh = 16.0 * s * np.sqrt(r)
            ww = 16.0 * s * np.sqrt(1.0 / r)
            base.append([-ww / 2.0, -hh / 2.0, ww / 2.0, hh / 2.0])
    base = np.asarray(base, np.float32)
    sy = (np.arange(_H, dtype=np.float32) + 0.5) * 16.0
    sx = (np.arange(_W, dtype=np.float32) + 0.5) * 16.0
    yy, xx = np.meshgrid(sy, sx, indexing="ij")
    shifts = np.stack([xx.ravel(), yy.ravel(), xx.ravel(), yy.ravel()], axis=1)
    anch = shifts[:, None, :] + base[None, :, :]
    x1, y1 = anch[..., 0], anch[..., 1]
    x2, y2 = anch[..., 2], anch[..., 3]
    aw = x2 - x1
    ah = y2 - y1
    acx = x1 + np.float32(0.5) * aw
    acy = y1 + np.float32(0.5) * ah

    def toq(a):
        out = np.zeros((_H, _QW, _NA), np.float32)
        out[:, :_W, :] = a.reshape(_H, _W, _NA)
        return out.reshape(_NQ, _NA)

    return jnp.asarray(np.concatenate([toq(acx), toq(acy), toq(aw), toq(ah)], axis=1))


def _rpn_head_kernel(x_ref, w9_ref, b1_ref, hw_ref, hb_ref, anc_ref, sz_ref,
                     score_ref, x1_ref, y1_ref, x2_ref, y2_ref, xp_ref):
    xp_ref[...] = jnp.zeros(xp_ref.shape, _F32)
    for r in range(_H):
        xp_ref[pl.ds((r + 1) * _QW + 1, _W), :] = x_ref[pl.ds(r * _W, _W), :]

    acc = jnp.zeros((_NQ, 512), _F32) + b1_ref[...]
    for ky in range(3):
        for kx in range(3):
            t = ky * 3 + kx
            off = ky * _QW + kx
            acc = acc + jnp.dot(xp_ref[pl.ds(off, _NQ), :], w9_ref[t],
                                preferred_element_type=_F32)
    mid = jnp.maximum(acc, 0.0)

    heads = jnp.dot(mid, hw_ref[...], preferred_element_type=_F32) + hb_ref[...]
    tx = heads[:, 0:9]
    ty = heads[:, 9:18]
    tw = heads[:, 18:27]
    th = heads[:, 27:36]
    s0 = heads[:, 36:45]
    s1 = heads[:, 45:54]

    acx = anc_ref[:, 0:9]
    acy = anc_ref[:, 9:18]
    aw = anc_ref[:, 18:27]
    ah = anc_ref[:, 27:36]

    cx = tx * aw + acx
    cy = ty * ah + acy
    w = jnp.exp(jnp.clip(tw, -10.0, 10.0)) * aw
    h = jnp.exp(jnp.clip(th, -10.0, 10.0)) * ah

    sz = sz_ref[...]
    bx1 = jnp.clip(cx - 0.5 * w, 0.0, sz)
    by1 = jnp.clip(cy - 0.5 * h, 0.0, sz)
    bx2 = jnp.clip(cx + 0.5 * w, 0.0, sz)
    by2 = jnp.clip(cy + 0.5 * h, 0.0, sz)

    m = jnp.maximum(s0, s1)
    e0 = jnp.exp(s0 - m)
    e1 = jnp.exp(s1 - m)
    fg = e1 / (e0 + e1)

    valid = ((bx2 - bx1) >= 16.0) & ((by2 - by1) >= 16.0)
    score_ref[...] = jnp.where(valid, fg, -1e9)
    x1_ref[...] = bx1
    y1_ref[...] = by1
    x2_ref[...] = bx2
    y2_ref[...] = by2


def _nms_kernel(cc_ref, cr_ref, out_ref, sup_ref):
    rx1 = cr_ref[0:1, :]
    ry1 = cr_ref[1:2, :]
    rx2 = cr_ref[2:3, :]
    ry2 = cr_ref[3:4, :]
    arear = (rx2 - rx1) * (ry2 - ry1)

    futsup = jnp.zeros((1, _PRE_PAD), _F32)
    for k in range(_NBLK):
        c0 = k * _BLK
        cb = cc_ref[pl.ds(c0, _BLK), :]
        cx1 = cb[:, 0:1]
        cy1 = cb[:, 1:2]
        cx2 = cb[:, 2:3]
        cy2 = cb[:, 3:4]
        areac = (cx2 - cx1) * (cy2 - cy1)
        xx1 = jnp.maximum(cx1, rx1)
        yy1 = jnp.maximum(cy1, ry1)
        xx2 = jnp.minimum(cx2, rx2)
        yy2 = jnp.minimum(cy2, ry2)
        inter = jnp.maximum(xx2 - xx1, 0.0) * jnp.maximum(yy2 - yy1, 0.0)
        iou = inter / jnp.maximum(areac + arear - inter, 1e-9)
        jj = lax.broadcasted_iota(_I32, (_BLK, _PRE_PAD), 1)
        ii = lax.broadcasted_iota(_I32, (_BLK, _PRE_PAD), 0) + c0
        srow = jnp.where((iou > _THRESH) & (jj > ii), 1.0, 0.0)

        diag = srow[:, c0:c0 + _BLK]
        ext = futsup[0:1, c0:c0 + _BLK]
        sup = ext
        for _ in range(_JACOBI):
            hit = jnp.dot(1.0 - sup, diag, preferred_element_type=_F32)
            sup = jnp.maximum(ext, jnp.where(hit > 0.0, 1.0, 0.0))
        hit = jnp.dot(1.0 - sup, diag, preferred_element_type=_F32)
        sup2 = jnp.maximum(ext, jnp.where(hit > 0.0, 1.0, 0.0))
        delta = jnp.sum(jnp.abs(sup2 - sup))

        @pl.when(delta == 0.0)
        def _():
            sup_ref[pl.ds(k, 1), :] = sup2

        @pl.when(delta != 0.0)
        def _():
            sblk = ext
            for j in range(_BLK):
                row = diag[j:j + 1, :]
                sj = sblk[0:1, j:j + 1]
                sblk = jnp.where(sj > 0.0, sblk, jnp.maximum(sblk, row))
            sup_ref[pl.ds(k, 1), :] = sblk

        sblk = sup_ref[pl.ds(k, 1), :]

        contrib = jnp.dot(1.0 - sblk, srow, preferred_element_type=_F32)
        futsup = jnp.maximum(futsup, jnp.where(contrib > 0.0, 1.0, 0.0))

    sup16 = sup_ref[...]
    fidx = (lax.broadcasted_iota(_I32, (_NBLK, _BLK), 0) * _BLK
            + lax.broadcasted_iota(_I32, (_NBLK, _BLK), 1))
    valid16 = jnp.where(fidx < _PRE, 1.0, 0.0)
    kept16 = (1.0 - sup16) * valid16
    supv16 = sup16 * valid16
    tri = jnp.where(
        lax.broadcasted_iota(_I32, (_BLK, _BLK), 0)
        <= lax.broadcasted_iota(_I32, (_BLK, _BLK), 1), 1.0, 0.0)
    ckr = jnp.dot(kept16, tri, preferred_element_type=_F32)
    csr = jnp.dot(supv16, tri, preferred_element_type=_F32)
    low = jnp.where(
        lax.broadcasted_iota(_I32, (_NBLK, _NBLK), 0)
        > lax.broadcasted_iota(_I32, (_NBLK, _NBLK), 1), 1.0, 0.0)
    ck = ckr + jnp.dot(low, ckr[:, _BLK - 1:_BLK], preferred_element_type=_F32)
    cs = csr + jnp.dot(low, csr[:, _BLK - 1:_BLK], preferred_element_type=_F32)
    nkept = jnp.sum(kept16)
    rank16 = (kept16 * (ck - 1.0) + supv16 * (nkept + cs - 1.0)
              + (1.0 - valid16) * 4096.0)
    rank = jnp.concatenate([rank16[k:k + 1, :] for k in range(_NBLK)], axis=1)

    rr = lax.broadcasted_iota(_I32, (_POST, _PRE_PAD), 0)
    onehot = jnp.where(
        jnp.broadcast_to(rank.astype(_I32), (_POST, _PRE_PAD)) == rr, 1.0, 0.0)
    out_ref[...] = jnp.dot(onehot, cc_ref[...], preferred_element_type=_F32)


def kernel(features, img_size, conv1_w, conv1_b, score_w, score_b, loc_w, loc_b):
    x = jnp.transpose(features[0], (1, 2, 0)).reshape(_NPIX, 512)
    w9 = jnp.transpose(conv1_w, (2, 3, 1, 0)).reshape(9, 512, 512)
    b1 = conv1_b.reshape(1, 512)
    lw = loc_w[:, :, 0, 0]
    sw = score_w[:, :, 0, 0]
    hw = jnp.concatenate(
        [lw[0::4].T, lw[1::4].T, lw[2::4].T, lw[3::4].T, sw[0::2].T, sw[1::2].T],
        axis=1)
    hb = jnp.concatenate(
        [loc_b[0::4], loc_b[1::4], loc_b[2::4], loc_b[3::4],
         score_b[0::2], score_b[1::2]]).reshape(1, 54)
    anc = _anchor_qconst()
    sz = jnp.asarray(img_size, _F32).reshape(1, 1)

    q9 = jax.ShapeDtypeStruct((_NQ, _NA), _F32)
    score_q, qx1, qy1, qx2, qy2 = pl.pallas_call(
        _rpn_head_kernel,
        out_shape=[q9, q9, q9, q9, q9],
        scratch_shapes=[pltpu.VMEM((_NQ + 120, 512), _F32)],
    )(x, w9, b1, hw, hb, anc, sz)

    def unq(a):
        return a.reshape(_H, _QW, _NA)[:, :_W, :].reshape(-1)

    scores = unq(score_q)
    boxes = jnp.stack([unq(qx1), unq(qy1), unq(qx2), unq(qy2)], axis=1)
    _, top_i = lax.top_k(scores, _PRE)
    cand = boxes[top_i]
    cc = jnp.zeros((_PRE_PAD, 4), _F32).at[:_PRE].set(cand)
    cr = cc.T

    out = pl.pallas_call(
        _nms_kernel,
        out_shape=jax.ShapeDtypeStruct((_POST, 4), _F32),
        scratch_shapes=[pltpu.VMEM((_NBLK, _BLK), _F32)],
    )(cc, cr)
    return out[None]
